# baseline (speedup 1.0000x reference)
import jax
import jax.numpy as jnp
from jax.experimental import pallas as pl

_E = 8
_T = 2048
_NS = 4


def _top2(logits):
    eidx = jax.lax.broadcasted_iota(jnp.int32, logits.shape, 0)
    m1 = jnp.max(logits, axis=0, keepdims=True)
    i1 = jnp.min(jnp.where(logits == m1, eidx, _E), axis=0, keepdims=True)
    masked = jnp.where(eidx == i1, -jnp.inf, logits)
    m2 = jnp.max(masked, axis=0, keepdims=True)
    i2 = jnp.min(jnp.where(masked == m2, eidx, _E), axis=0, keepdims=True)
    w1 = 1.0 / (1.0 + jnp.exp(m2 - m1))
    return (jnp.concatenate([i1, i2], axis=0),
            jnp.concatenate([w1, 1.0 - w1], axis=0))


def _router_body(*refs):
    x_refs = refs[:_NS]
    w_ref = refs[_NS]
    idx_ref, wgt_ref = refs[_NS + 1], refs[_NS + 2]
    w = w_ref[...]
    dn = (((1,), (1,)), ((), ()))
    for k in range(_NS):
        lg = jax.lax.dot_general(w, x_refs[k][...], dn,
                                 preferred_element_type=jnp.float32)
        i_k, g_k = _top2(lg)
        idx_ref[:, k * _T:(k + 1) * _T] = i_k
        wgt_ref[:, k * _T:(k + 1) * _T] = g_k


def kernel(hidden_states, weights):
    b, s, d = hidden_states.shape
    n = b * s
    hs = hidden_states.reshape(n, d)
    nblk = n // _T

    def make_in_spec(k):
        return pl.BlockSpec((_T, d), lambda i, k=k: (_NS * i + k, 0))

    idx_t, wgt_t = pl.pallas_call(
        _router_body,
        grid=(nblk // _NS,),
        in_specs=[make_in_spec(k) for k in range(_NS)]
        + [pl.BlockSpec((_E, d), lambda i: (0, 0))],
        out_specs=[
            pl.BlockSpec((2, _NS * _T), lambda i: (0, i)),
            pl.BlockSpec((2, _NS * _T), lambda i: (0, i)),
        ],
        out_shape=[
            jax.ShapeDtypeStruct((2, n), jnp.int32),
            jax.ShapeDtypeStruct((2, n), jnp.float32),
        ],
    )(*([hs] * _NS + [weights]))
    return idx_t.T, wgt_t.T, jnp.float32(0.0)


# speedup vs baseline: 1.0694x; 1.0694x over previous
import jax
import jax.numpy as jnp
from jax.experimental import pallas as pl

_E = 8
_T = 1024
_NS = 4


def _top2(logits):
    eidx = jax.lax.broadcasted_iota(jnp.int32, logits.shape, 0)
    m1 = jnp.max(logits, axis=0, keepdims=True)
    i1 = jnp.min(jnp.where(logits == m1, eidx, _E), axis=0, keepdims=True)
    masked = jnp.where(eidx == i1, -jnp.inf, logits)
    m2 = jnp.max(masked, axis=0, keepdims=True)
    i2 = jnp.min(jnp.where(masked == m2, eidx, _E), axis=0, keepdims=True)
    w1 = 1.0 / (1.0 + jnp.exp(m2 - m1))
    return (jnp.concatenate([i1, i2], axis=0),
            jnp.concatenate([w1, 1.0 - w1], axis=0))


def _router_body(*refs):
    x_refs = refs[:_NS]
    w_ref = refs[_NS]
    idx_ref, wgt_ref = refs[_NS + 1], refs[_NS + 2]
    w = w_ref[...]
    dn = (((1,), (1,)), ((), ()))
    for k in range(_NS):
        lg = jax.lax.dot_general(w, x_refs[k][...], dn,
                                 preferred_element_type=jnp.float32)
        i_k, g_k = _top2(lg)
        idx_ref[:, k * _T:(k + 1) * _T] = i_k
        wgt_ref[:, k * _T:(k + 1) * _T] = g_k


def kernel(hidden_states, weights):
    b, s, d = hidden_states.shape
    n = b * s
    hs = hidden_states.reshape(n, d)
    nblk = n // _T

    def make_in_spec(k):
        return pl.BlockSpec((_T, d), lambda i, k=k: (_NS * i + k, 0))

    idx_t, wgt_t = pl.pallas_call(
        _router_body,
        grid=(nblk // _NS,),
        in_specs=[make_in_spec(k) for k in range(_NS)]
        + [pl.BlockSpec((_E, d), lambda i: (0, 0))],
        out_specs=[
            pl.BlockSpec((2, _NS * _T), lambda i: (0, i)),
            pl.BlockSpec((2, _NS * _T), lambda i: (0, i)),
        ],
        out_shape=[
            jax.ShapeDtypeStruct((2, n), jnp.int32),
            jax.ShapeDtypeStruct((2, n), jnp.float32),
        ],
    )(*([hs] * _NS + [weights]))
    return idx_t.T, wgt_t.T, jnp.float32(0.0)
